# baseline (speedup 1.0000x reference)
import functools

import jax
import jax.numpy as jnp
from jax import lax
from jax.experimental import pallas as pl
from jax.experimental.pallas import tpu as pltpu
from jax.experimental.pallas import tpu_sc as plsc

N = 10000
E = 320000
D = 128
G = 64
C = 10
EPS_BN = 1e-5

NPAD = 10112
ROWS_PER_TILE = NPAD // 16
B = 128
CPT = 81
NTILES = 32
EPAD = NTILES * CPT * B
NRING = 3

BLK = 400
NBLK = N // BLK

_SC_MESH = plsc.VectorSubcoreMesh(core_axis_name="c", subcore_axis_name="s")


@functools.partial(
    pl.kernel,
    out_type=jax.ShapeDtypeStruct((NTILES, NPAD), jnp.float32),
    mesh=_SC_MESH,
    scratch_types=[
        pltpu.VMEM((CPT * B,), jnp.int32),
        pltpu.VMEM((NPAD,), jnp.float32),
    ],
    compiler_params=pltpu.CompilerParams(needs_layout_passes=False),
)
def _sc_hist(dst1d_hbm, out_hbm, didx_ref, hist_ref):
    cid = lax.axis_index("c")
    sid = lax.axis_index("s")
    w = cid * 16 + sid

    @pl.loop(0, NPAD // 16)
    def _zero(i):
        hist_ref[pl.ds(i * 16, 16)] = jnp.zeros((16,), jnp.float32)

    pltpu.sync_copy(dst1d_hbm.at[pl.ds(w * (CPT * B), CPT * B)], didx_ref)
    ones = jnp.ones((16,), jnp.float32)

    @pl.loop(0, CPT * B // 16)
    def _vec(c):
        idx = didx_ref[pl.ds(c * 16, 16)]
        plsc.addupdate_scatter(hist_ref, [idx], ones)

    pltpu.sync_copy(hist_ref, out_hbm.at[w])


@functools.partial(
    pl.kernel,
    out_type=jax.ShapeDtypeStruct((2, NPAD, D), jnp.float32),
    mesh=_SC_MESH,
    scratch_types=[
        pltpu.VMEM((B,), jnp.int32),
        pltpu.VMEM((B,), jnp.int32),
        pltpu.VMEM((B,), jnp.int32),
        pltpu.VMEM((B,), jnp.int32),
        pltpu.VMEM((B,), jnp.int32),
        pltpu.VMEM((B,), jnp.int32),
        pltpu.VMEM((B, D), jnp.float32),
        pltpu.VMEM((B, D), jnp.float32),
        pltpu.VMEM((B, D), jnp.float32),
        pltpu.VMEM_SHARED((NPAD, D), jnp.float32),
        pltpu.SemaphoreType.DMA,
        pltpu.SemaphoreType.DMA,
        pltpu.SemaphoreType.DMA,
        pltpu.SemaphoreType.DMA,
        pltpu.SemaphoreType.DMA,
        pltpu.SemaphoreType.DMA,
        pltpu.SemaphoreType.DMA,
        pltpu.SemaphoreType.DMA,
        pltpu.SemaphoreType.DMA,
    ],
    compiler_params=pltpu.CompilerParams(needs_layout_passes=False),
)
def _sc_scatter(src1d, dst1d, hs, zeros, parts,
                sidx0, sidx1, sidx2, didx0, didx1, didx2,
                rows0, rows1, rows2, acc,
                isem0, isem1, isem2, gsem0, gsem1, gsem2,
                ssem0, ssem1, ssem2):
    cid = lax.axis_index("c")
    sid = lax.axis_index("s")
    w = cid * 16 + sid
    base = w * CPT

    sidx = (sidx0, sidx1, sidx2)
    didx = (didx0, didx1, didx2)
    rows = (rows0, rows1, rows2)
    isems = (isem0, isem1, isem2)
    gsems = (gsem0, gsem1, gsem2)
    ssems = (ssem0, ssem1, ssem2)

    def load_idx(c, b):
        pltpu.async_copy(src1d.at[pl.ds((base + c) * B, B)], sidx[b], isems[b])
        pltpu.async_copy(dst1d.at[pl.ds((base + c) * B, B)], didx[b], isems[b])

    def wait_idx(b):
        pltpu.make_async_copy(src1d.at[pl.ds(0, B)], sidx[b], isems[b]).wait()
        pltpu.make_async_copy(dst1d.at[pl.ds(0, B)], didx[b], isems[b]).wait()

    load_idx(0, 0)
    load_idx(1, 1)

    pltpu.sync_copy(zeros.at[pl.ds(sid * ROWS_PER_TILE, ROWS_PER_TILE)],
                    acc.at[pl.ds(sid * ROWS_PER_TILE, ROWS_PER_TILE)])
    wait_idx(0)
    pltpu.async_copy(hs.at[sidx0], rows0, gsem0)
    plsc.subcore_barrier()

    @pl.loop(0, CPT, step=NRING)
    def _chunk(c):
        for b in range(NRING):
            cc = c + b
            b1 = (b + 1) % NRING
            b2 = (b + 2) % NRING
            pltpu.make_async_copy(hs.at[sidx[b]], rows[b], gsems[b]).wait()
            pltpu.async_copy(rows[b], acc.at[didx[b]], ssems[b], add=True)

            @pl.when(cc + 1 < CPT)
            def _next_gather():
                wait_idx(b1)
                pltpu.async_copy(hs.at[sidx[b1]], rows[b1], gsems[b1])

            @pl.when(cc + 2 < CPT)
            def _next_idx():
                @pl.when(cc >= 1)
                def _free_slot():
                    pltpu.make_async_copy(rows[b2], acc.at[didx[b2]],
                                          ssems[b2]).wait()

                load_idx(cc + 2, b2)

    for b in range(NRING):
        pltpu.make_async_copy(rows[b], acc.at[didx[b]], ssems[b]).wait()

    plsc.subcore_barrier()
    pltpu.sync_copy(acc.at[pl.ds(sid * ROWS_PER_TILE, ROWS_PER_TILE)],
                    parts.at[cid, pl.ds(sid * ROWS_PER_TILE, ROWS_PER_TILE)])


def _t0_body(h_ref, o_ref):
    s = jnp.sum(h_ref[...], axis=0, keepdims=True)
    o_ref[...] = lax.rsqrt(s + 1.0)


_t0 = pl.pallas_call(
    _t0_body,
    grid=(1,),
    in_specs=[pl.BlockSpec((NTILES, NPAD), lambda i: (0, 0))],
    out_specs=pl.BlockSpec((1, NPAD), lambda i: (0, 0)),
    out_shape=jax.ShapeDtypeStruct((1, NPAD), jnp.float32),
)


def _t1_body(x_ref, w_ref, dv_ref, o_ref):
    o_ref[...] = jnp.dot(x_ref[...], w_ref[...],
                         preferred_element_type=jnp.float32) * dv_ref[...]


_t1 = pl.pallas_call(
    _t1_body,
    grid=(NBLK,),
    in_specs=[
        pl.BlockSpec((BLK, D), lambda i: (i, 0)),
        pl.BlockSpec((D, D), lambda i: (0, 0)),
        pl.BlockSpec((BLK, 1), lambda i: (i, 0)),
    ],
    out_specs=pl.BlockSpec((BLK, D), lambda i: (i, 0)),
    out_shape=jax.ShapeDtypeStruct((N, D), jnp.float32),
)


def _t2_body(p0_ref, p1_ref, hs_ref, dv_ref, z_ref, st_ref, s1, s2):
    i = pl.program_id(0)
    z = (p0_ref[0] + p1_ref[0] + hs_ref[...]) * dv_ref[...]
    z_ref[...] = z
    c1 = jnp.sum(z, axis=0, keepdims=True)
    c2 = jnp.sum(z * z, axis=0, keepdims=True)

    @pl.when(i == 0)
    def _init():
        s1[...] = c1
        s2[...] = c2

    @pl.when(i > 0)
    def _acc():
        s1[...] += c1
        s2[...] += c2

    @pl.when(i == NBLK - 1)
    def _fin():
        st_ref[...] = jnp.concatenate([s1[...], s2[...]], axis=0)


_t2 = pl.pallas_call(
    _t2_body,
    grid=(NBLK,),
    in_specs=[
        pl.BlockSpec((1, BLK, D), lambda i: (0, i, 0)),
        pl.BlockSpec((1, BLK, D), lambda i: (1, i, 0)),
        pl.BlockSpec((BLK, D), lambda i: (i, 0)),
        pl.BlockSpec((BLK, 1), lambda i: (i, 0)),
    ],
    out_specs=[
        pl.BlockSpec((BLK, D), lambda i: (i, 0)),
        pl.BlockSpec((2, D), lambda i: (0, 0)),
    ],
    out_shape=[
        jax.ShapeDtypeStruct((N, D), jnp.float32),
        jax.ShapeDtypeStruct((2, D), jnp.float32),
    ],
    scratch_shapes=[
        pltpu.VMEM((1, D), jnp.float32),
        pltpu.VMEM((1, D), jnp.float32),
    ],
)


def _bn_relu(z, st, g, be):
    mean = st[0:1] * (1.0 / N)
    var = st[1:2] * (1.0 / N) - mean * mean
    rstd = lax.rsqrt(var + EPS_BN)
    return jnp.maximum((z - mean) * (rstd * g) + be, 0.0)


def _t3_body(z_ref, st_ref, g_ref, be_ref, w_ref, dv_ref, o_ref):
    h = _bn_relu(z_ref[...], st_ref[...], g_ref[...], be_ref[...])
    o_ref[...] = jnp.dot(h, w_ref[...],
                         preferred_element_type=jnp.float32) * dv_ref[...]


_t3 = pl.pallas_call(
    _t3_body,
    grid=(NBLK,),
    in_specs=[
        pl.BlockSpec((BLK, D), lambda i: (i, 0)),
        pl.BlockSpec((2, D), lambda i: (0, 0)),
        pl.BlockSpec((1, D), lambda i: (0, 0)),
        pl.BlockSpec((1, D), lambda i: (0, 0)),
        pl.BlockSpec((D, D), lambda i: (0, 0)),
        pl.BlockSpec((BLK, 1), lambda i: (i, 0)),
    ],
    out_specs=pl.BlockSpec((BLK, D), lambda i: (i, 0)),
    out_shape=jax.ShapeDtypeStruct((N, D), jnp.float32),
)


def _t4_body(z_ref, st_ref, g_ref, be_ref, b_ref, wc_ref, bc_ref,
             feat_ref, pred_ref, pool, cnt):
    i = pl.program_id(0)
    h = _bn_relu(z_ref[...], st_ref[...], g_ref[...], be_ref[...])
    ids = lax.broadcasted_iota(jnp.int32, (BLK, G), 1)
    oh = (b_ref[...] == ids).astype(jnp.float32)
    ppart = lax.dot_general(oh, h, (((0,), (0,)), ((), ())),
                            preferred_element_type=jnp.float32)
    cpart = lax.dot_general(oh, jnp.ones((BLK, D), jnp.float32),
                            (((0,), (0,)), ((), ())),
                            preferred_element_type=jnp.float32)

    @pl.when(i == 0)
    def _init():
        pool[...] = ppart
        cnt[...] = cpart

    @pl.when(i > 0)
    def _acc():
        pool[...] += ppart
        cnt[...] += cpart

    @pl.when(i == NBLK - 1)
    def _fin():
        feat = pool[...] / jnp.maximum(cnt[...], 1.0)
        nrm = jnp.sqrt(jnp.sum(feat * feat, axis=1, keepdims=True))
        feat = feat / jnp.maximum(nrm, 1e-12)
        feat_ref[...] = feat
        pred_ref[...] = jnp.dot(feat, wc_ref[...],
                                preferred_element_type=jnp.float32) + bc_ref[...]


_t4 = pl.pallas_call(
    _t4_body,
    grid=(NBLK,),
    in_specs=[
        pl.BlockSpec((BLK, D), lambda i: (i, 0)),
        pl.BlockSpec((2, D), lambda i: (0, 0)),
        pl.BlockSpec((1, D), lambda i: (0, 0)),
        pl.BlockSpec((1, D), lambda i: (0, 0)),
        pl.BlockSpec((BLK, 1), lambda i: (i, 0)),
        pl.BlockSpec((D, D), lambda i: (0, 0)),
        pl.BlockSpec((1, D), lambda i: (0, 0)),
    ],
    out_specs=[
        pl.BlockSpec((G, D), lambda i: (0, 0)),
        pl.BlockSpec((G, D), lambda i: (0, 0)),
    ],
    out_shape=[
        jax.ShapeDtypeStruct((G, D), jnp.float32),
        jax.ShapeDtypeStruct((G, D), jnp.float32),
    ],
    scratch_shapes=[
        pltpu.VMEM((G, D), jnp.float32),
        pltpu.VMEM((G, D), jnp.float32),
    ],
)


def kernel(x, edge_index, batch, W1, b1, g1, be1, W2, b2, g2, be2,
           W3, b3, g3, be3, Wc, bc):
    f32 = jnp.float32
    pad = EPAD - E
    pad_i = jnp.arange(pad, dtype=jnp.int32)
    src1d = jnp.concatenate([edge_index[0], pad_i % N])
    dst1d = jnp.concatenate([edge_index[1], N + (pad_i % (NPAD - N))])
    zeros_np = jnp.zeros((NPAD, D), f32)

    hists = _sc_hist(dst1d)
    dinv_col = _t0(hists).reshape(NPAD, 1)

    g1r, be1r = g1.reshape(1, D), be1.reshape(1, D)
    g2r, be2r = g2.reshape(1, D), be2.reshape(1, D)
    g3r, be3r = g3.reshape(1, D), be3.reshape(1, D)
    batch_col = batch.reshape(N, 1)
    Wc_pad = jnp.zeros((D, D), f32).at[:, :C].set(Wc)
    bc_pad = jnp.zeros((1, D), f32).at[0, :C].set(bc)

    hs = _t1(x, W1, dinv_col)
    parts = _sc_scatter(src1d, dst1d, hs, zeros_np)
    z, st = _t2(parts, parts, hs, dinv_col)

    hs = _t3(z, st, g1r, be1r, W2, dinv_col)
    parts = _sc_scatter(src1d, dst1d, hs, zeros_np)
    z, st = _t2(parts, parts, hs, dinv_col)

    hs = _t3(z, st, g2r, be2r, W3, dinv_col)
    parts = _sc_scatter(src1d, dst1d, hs, zeros_np)
    z, st = _t2(parts, parts, hs, dinv_col)

    feat, pred_pad = _t4(z, st, g3r, be3r, batch_col, Wc_pad, bc_pad)
    return (pred_pad[:, :C], feat)


# speedup vs baseline: 1.2349x; 1.2349x over previous
import functools

import jax
import jax.numpy as jnp
from jax import lax
from jax.experimental import pallas as pl
from jax.experimental.pallas import tpu as pltpu
from jax.experimental.pallas import tpu_sc as plsc

N = 10000
E = 320000
D = 128
G = 64
C = 10
EPS_BN = 1e-5

NPAD = 10240
ROWS_PER_TILE = NPAD // 16
B = 128
CPT = 80
PHASES = 2
CPP = CPT // PHASES
NTILES = 32
EPAD = NTILES * CPT * B

BLK = 1000
NBLK = N // BLK

_SC_MESH = plsc.VectorSubcoreMesh(core_axis_name="c", subcore_axis_name="s")


@functools.partial(
    pl.kernel,
    out_type=jax.ShapeDtypeStruct((NTILES, NPAD), jnp.float32),
    mesh=_SC_MESH,
    scratch_types=[
        pltpu.VMEM((CPT * B,), jnp.int32),
        pltpu.VMEM((NPAD,), jnp.float32),
    ],
    compiler_params=pltpu.CompilerParams(needs_layout_passes=False),
)
def _sc_hist(dst1d_hbm, out_hbm, didx_ref, hist_ref):
    cid = lax.axis_index("c")
    sid = lax.axis_index("s")
    w = cid * 16 + sid

    @pl.loop(0, NPAD // 16)
    def _zero(i):
        hist_ref[pl.ds(i * 16, 16)] = jnp.zeros((16,), jnp.float32)

    pltpu.sync_copy(dst1d_hbm.at[pl.ds(w * (CPT * B), CPT * B)], didx_ref)
    ones = jnp.ones((16,), jnp.float32)

    @pl.loop(0, CPT * B // 16)
    def _vec(c):
        idx = didx_ref[pl.ds(c * 16, 16)]
        plsc.addupdate_scatter(hist_ref, [idx], ones)

    pltpu.sync_copy(hist_ref, out_hbm.at[w])


@functools.partial(
    pl.kernel,
    out_type=jax.ShapeDtypeStruct((2, NPAD, D), jnp.float32),
    mesh=_SC_MESH,
    scratch_types=[
        pltpu.VMEM((CPP, B), jnp.int32),
        pltpu.VMEM((CPP, B), jnp.int32),
        pltpu.VMEM((B, D), jnp.float32),
        pltpu.VMEM((B, D), jnp.float32),
        pltpu.VMEM_SHARED((NPAD, D), jnp.float32),
        pltpu.SemaphoreType.DMA,
        pltpu.SemaphoreType.DMA,
    ],
    compiler_params=pltpu.CompilerParams(needs_layout_passes=False),
)
def _sc_scatter(src2d, dst2d, hs, zeros, parts, sidx, didx, rows0, rows1,
                acc, sem0, sem1):
    cid = lax.axis_index("c")
    sid = lax.axis_index("s")
    w = cid * 16 + sid

    rows = (rows0, rows1)
    sems = (sem0, sem1)

    r0 = sid * ROWS_PER_TILE

    @pl.when((cid == 0) & (sid < 15))
    def _init_hs():
        pltpu.sync_copy(hs.at[pl.ds(r0, ROWS_PER_TILE)],
                        acc.at[pl.ds(r0, ROWS_PER_TILE)])

    @pl.when((cid == 0) & (sid == 15))
    def _init_hs_tail():
        pltpu.sync_copy(hs.at[pl.ds(15 * ROWS_PER_TILE, N - 15 * ROWS_PER_TILE)],
                        acc.at[pl.ds(15 * ROWS_PER_TILE, N - 15 * ROWS_PER_TILE)])
        pltpu.sync_copy(zeros.at[pl.ds(N, NPAD - N)],
                        acc.at[pl.ds(N, NPAD - N)])

    @pl.when(cid == 1)
    def _init_zero():
        pltpu.sync_copy(zeros.at[pl.ds(r0, ROWS_PER_TILE)],
                        acc.at[pl.ds(r0, ROWS_PER_TILE)])

    plsc.subcore_barrier()

    @pl.loop(0, PHASES)
    def _phase(p):
        base = w * CPT + p * CPP
        pltpu.sync_copy(src2d.at[pl.ds(base, CPP)], sidx)
        pltpu.sync_copy(dst2d.at[pl.ds(base, CPP)], didx)
        pltpu.async_copy(hs.at[sidx.at[0]], rows0, sem0)
        pltpu.async_copy(hs.at[sidx.at[1]], rows1, sem1)

        @pl.loop(0, CPP, step=2)
        def _chunk(c):
            for b in range(2):
                cc = c + b
                pltpu.make_async_copy(hs.at[sidx.at[cc]], rows[b],
                                      sems[b]).wait()
                pltpu.sync_copy(rows[b], acc.at[didx.at[cc]], add=True)

                @pl.when(cc + 2 < CPP)
                def _refill():
                    pltpu.async_copy(hs.at[sidx.at[cc + 2]], rows[b], sems[b])

    plsc.subcore_barrier()
    pltpu.sync_copy(acc.at[pl.ds(sid * ROWS_PER_TILE, ROWS_PER_TILE)],
                    parts.at[cid, pl.ds(sid * ROWS_PER_TILE, ROWS_PER_TILE)])


def _t0_body(h_ref, o_ref):
    s = jnp.sum(h_ref[...], axis=0, keepdims=True)
    o_ref[...] = lax.rsqrt(s + 1.0)


_t0 = pl.pallas_call(
    _t0_body,
    grid=(1,),
    in_specs=[pl.BlockSpec((NTILES, NPAD), lambda i: (0, 0))],
    out_specs=pl.BlockSpec((1, NPAD), lambda i: (0, 0)),
    out_shape=jax.ShapeDtypeStruct((1, NPAD), jnp.float32),
)


def _t1_body(x_ref, w_ref, dv_ref, o_ref):
    o_ref[...] = jnp.dot(x_ref[...], w_ref[...],
                         preferred_element_type=jnp.float32) * dv_ref[...]


_t1 = pl.pallas_call(
    _t1_body,
    grid=(NBLK,),
    in_specs=[
        pl.BlockSpec((BLK, D), lambda i: (i, 0)),
        pl.BlockSpec((D, D), lambda i: (0, 0)),
        pl.BlockSpec((BLK, 1), lambda i: (i, 0)),
    ],
    out_specs=pl.BlockSpec((BLK, D), lambda i: (i, 0)),
    out_shape=jax.ShapeDtypeStruct((N, D), jnp.float32),
)


def _t2_body(p0_ref, p1_ref, dv_ref, z_ref, st_ref, s1, s2):
    i = pl.program_id(0)
    z = (p0_ref[0] + p1_ref[0]) * dv_ref[...]
    z_ref[...] = z
    c1 = jnp.sum(z, axis=0, keepdims=True)
    c2 = jnp.sum(z * z, axis=0, keepdims=True)

    @pl.when(i == 0)
    def _init():
        s1[...] = c1
        s2[...] = c2

    @pl.when(i > 0)
    def _acc():
        s1[...] += c1
        s2[...] += c2

    @pl.when(i == NBLK - 1)
    def _fin():
        st_ref[...] = jnp.concatenate([s1[...], s2[...]], axis=0)


_t2 = pl.pallas_call(
    _t2_body,
    grid=(NBLK,),
    in_specs=[
        pl.BlockSpec((1, BLK, D), lambda i: (0, i, 0)),
        pl.BlockSpec((1, BLK, D), lambda i: (1, i, 0)),
        pl.BlockSpec((BLK, 1), lambda i: (i, 0)),
    ],
    out_specs=[
        pl.BlockSpec((BLK, D), lambda i: (i, 0)),
        pl.BlockSpec((2, D), lambda i: (0, 0)),
    ],
    out_shape=[
        jax.ShapeDtypeStruct((N, D), jnp.float32),
        jax.ShapeDtypeStruct((2, D), jnp.float32),
    ],
    scratch_shapes=[
        pltpu.VMEM((1, D), jnp.float32),
        pltpu.VMEM((1, D), jnp.float32),
    ],
)


def _bn_relu(z, st, g, be):
    mean = st[0:1] * (1.0 / N)
    var = st[1:2] * (1.0 / N) - mean * mean
    rstd = lax.rsqrt(var + EPS_BN)
    return jnp.maximum((z - mean) * (rstd * g) + be, 0.0)


def _t3_body(z_ref, st_ref, g_ref, be_ref, w_ref, dv_ref, o_ref):
    h = _bn_relu(z_ref[...], st_ref[...], g_ref[...], be_ref[...])
    o_ref[...] = jnp.dot(h, w_ref[...],
                         preferred_element_type=jnp.float32) * dv_ref[...]


_t3 = pl.pallas_call(
    _t3_body,
    grid=(NBLK,),
    in_specs=[
        pl.BlockSpec((BLK, D), lambda i: (i, 0)),
        pl.BlockSpec((2, D), lambda i: (0, 0)),
        pl.BlockSpec((1, D), lambda i: (0, 0)),
        pl.BlockSpec((1, D), lambda i: (0, 0)),
        pl.BlockSpec((D, D), lambda i: (0, 0)),
        pl.BlockSpec((BLK, 1), lambda i: (i, 0)),
    ],
    out_specs=pl.BlockSpec((BLK, D), lambda i: (i, 0)),
    out_shape=jax.ShapeDtypeStruct((N, D), jnp.float32),
)


def _t4_body(z_ref, st_ref, g_ref, be_ref, b_ref, wc_ref, bc_ref,
             feat_ref, pred_ref, pool, cnt):
    i = pl.program_id(0)
    h = _bn_relu(z_ref[...], st_ref[...], g_ref[...], be_ref[...])
    ids = lax.broadcasted_iota(jnp.int32, (BLK, G), 1)
    oh = (b_ref[...] == ids).astype(jnp.float32)
    ppart = lax.dot_general(oh, h, (((0,), (0,)), ((), ())),
                            preferred_element_type=jnp.float32)
    cpart = lax.dot_general(oh, jnp.ones((BLK, D), jnp.float32),
                            (((0,), (0,)), ((), ())),
                            preferred_element_type=jnp.float32)

    @pl.when(i == 0)
    def _init():
        pool[...] = ppart
        cnt[...] = cpart

    @pl.when(i > 0)
    def _acc():
        pool[...] += ppart
        cnt[...] += cpart

    @pl.when(i == NBLK - 1)
    def _fin():
        feat = pool[...] / jnp.maximum(cnt[...], 1.0)
        nrm = jnp.sqrt(jnp.sum(feat * feat, axis=1, keepdims=True))
        feat = feat / jnp.maximum(nrm, 1e-12)
        feat_ref[...] = feat
        pred_ref[...] = jnp.dot(feat, wc_ref[...],
                                preferred_element_type=jnp.float32) + bc_ref[...]


_t4 = pl.pallas_call(
    _t4_body,
    grid=(NBLK,),
    in_specs=[
        pl.BlockSpec((BLK, D), lambda i: (i, 0)),
        pl.BlockSpec((2, D), lambda i: (0, 0)),
        pl.BlockSpec((1, D), lambda i: (0, 0)),
        pl.BlockSpec((1, D), lambda i: (0, 0)),
        pl.BlockSpec((BLK, 1), lambda i: (i, 0)),
        pl.BlockSpec((D, D), lambda i: (0, 0)),
        pl.BlockSpec((1, D), lambda i: (0, 0)),
    ],
    out_specs=[
        pl.BlockSpec((G, D), lambda i: (0, 0)),
        pl.BlockSpec((G, D), lambda i: (0, 0)),
    ],
    out_shape=[
        jax.ShapeDtypeStruct((G, D), jnp.float32),
        jax.ShapeDtypeStruct((G, D), jnp.float32),
    ],
    scratch_shapes=[
        pltpu.VMEM((G, D), jnp.float32),
        pltpu.VMEM((G, D), jnp.float32),
    ],
)


def kernel(x, edge_index, batch, W1, b1, g1, be1, W2, b2, g2, be2,
           W3, b3, g3, be3, Wc, bc):
    f32 = jnp.float32
    pad = EPAD - E
    pad_i = jnp.arange(pad, dtype=jnp.int32)
    src1d = jnp.concatenate([edge_index[0], pad_i % N])
    dst1d = jnp.concatenate([edge_index[1], N + (pad_i % (NPAD - N))])
    src2d = src1d.reshape(EPAD // B, B)
    dst2d = dst1d.reshape(EPAD // B, B)
    zeros_np = jnp.zeros((NPAD, D), f32)

    hists = _sc_hist(dst1d)
    dinv_col = _t0(hists).reshape(NPAD, 1)

    g1r, be1r = g1.reshape(1, D), be1.reshape(1, D)
    g2r, be2r = g2.reshape(1, D), be2.reshape(1, D)
    g3r, be3r = g3.reshape(1, D), be3.reshape(1, D)
    batch_col = batch.reshape(N, 1)
    Wc_pad = jnp.zeros((D, D), f32).at[:, :C].set(Wc)
    bc_pad = jnp.zeros((1, D), f32).at[0, :C].set(bc)

    hs = _t1(x, W1, dinv_col)
    parts = _sc_scatter(src2d, dst2d, hs, zeros_np)
    z, st = _t2(parts, parts, dinv_col)

    hs = _t3(z, st, g1r, be1r, W2, dinv_col)
    parts = _sc_scatter(src2d, dst2d, hs, zeros_np)
    z, st = _t2(parts, parts, dinv_col)

    hs = _t3(z, st, g2r, be2r, W3, dinv_col)
    parts = _sc_scatter(src2d, dst2d, hs, zeros_np)
    z, st = _t2(parts, parts, dinv_col)

    feat, pred_pad = _t4(z, st, g3r, be3r, batch_col, Wc_pad, bc_pad)
    return (pred_pad[:, :C], feat)


# speedup vs baseline: 1.2569x; 1.0178x over previous
import functools

import jax
import jax.numpy as jnp
from jax import lax
from jax.experimental import pallas as pl
from jax.experimental.pallas import tpu as pltpu
from jax.experimental.pallas import tpu_sc as plsc

N = 10000
E = 320000
D = 128
G = 64
C = 10
EPS_BN = 1e-5

NPAD = 10240
ROWS_PER_TILE = NPAD // 16
B = 128
CPT = 80
PHASES = 2
CPP = CPT // PHASES
NTILES = 32
EPAD = NTILES * CPT * B

BLK = 1000
NBLK = N // BLK

_SC_MESH = plsc.VectorSubcoreMesh(core_axis_name="c", subcore_axis_name="s")


@functools.partial(
    pl.kernel,
    out_type=jax.ShapeDtypeStruct((NTILES, NPAD), jnp.float32),
    mesh=_SC_MESH,
    scratch_types=[
        pltpu.VMEM((CPT * B,), jnp.int32),
        pltpu.VMEM((NPAD,), jnp.float32),
    ],
    compiler_params=pltpu.CompilerParams(needs_layout_passes=False),
)
def _sc_hist(dst1d_hbm, out_hbm, didx_ref, hist_ref):
    cid = lax.axis_index("c")
    sid = lax.axis_index("s")
    w = cid * 16 + sid

    @pl.loop(0, NPAD // 16)
    def _zero(i):
        hist_ref[pl.ds(i * 16, 16)] = jnp.zeros((16,), jnp.float32)

    pltpu.sync_copy(dst1d_hbm.at[pl.ds(w * (CPT * B), CPT * B)], didx_ref)
    ones = jnp.ones((16,), jnp.float32)

    @pl.loop(0, CPT * B // 16)
    def _vec(c):
        idx = didx_ref[pl.ds(c * 16, 16)]
        plsc.addupdate_scatter(hist_ref, [idx], ones)

    pltpu.sync_copy(hist_ref, out_hbm.at[w])


@functools.partial(
    pl.kernel,
    out_type=jax.ShapeDtypeStruct((2, NPAD, D), jnp.float32),
    mesh=_SC_MESH,
    scratch_types=[
        pltpu.VMEM((CPP, B), jnp.int32),
        pltpu.VMEM((CPP, B), jnp.int32),
        pltpu.VMEM((B, D), jnp.float32),
        pltpu.VMEM((B, D), jnp.float32),
        pltpu.VMEM_SHARED((NPAD, D), jnp.float32),
        pltpu.SemaphoreType.DMA,
        pltpu.SemaphoreType.DMA,
    ],
    compiler_params=pltpu.CompilerParams(needs_layout_passes=False),
)
def _sc_scatter(src2d, dst2d, hs, zeros, parts, sidx, didx, rows0, rows1,
                acc, sem0, sem1):
    cid = lax.axis_index("c")
    sid = lax.axis_index("s")
    w = cid * 16 + sid

    rows = (rows0, rows1)
    sems = (sem0, sem1)

    r0 = sid * ROWS_PER_TILE

    @pl.when((cid == 0) & (sid < 15))
    def _init_hs():
        pltpu.sync_copy(hs.at[pl.ds(r0, ROWS_PER_TILE)],
                        acc.at[pl.ds(r0, ROWS_PER_TILE)])

    @pl.when((cid == 0) & (sid == 15))
    def _init_hs_tail():
        pltpu.sync_copy(hs.at[pl.ds(15 * ROWS_PER_TILE, N - 15 * ROWS_PER_TILE)],
                        acc.at[pl.ds(15 * ROWS_PER_TILE, N - 15 * ROWS_PER_TILE)])
        pltpu.sync_copy(zeros.at[pl.ds(N, NPAD - N)],
                        acc.at[pl.ds(N, NPAD - N)])

    @pl.when(cid == 1)
    def _init_zero():
        pltpu.sync_copy(zeros.at[pl.ds(r0, ROWS_PER_TILE)],
                        acc.at[pl.ds(r0, ROWS_PER_TILE)])

    plsc.subcore_barrier()

    @pl.loop(0, PHASES)
    def _phase(p):
        base = w * CPT + p * CPP
        pltpu.sync_copy(src2d.at[pl.ds(base, CPP)], sidx)
        pltpu.sync_copy(dst2d.at[pl.ds(base, CPP)], didx)
        pltpu.async_copy(hs.at[sidx.at[0]], rows0, sem0)
        pltpu.async_copy(hs.at[sidx.at[1]], rows1, sem1)

        @pl.loop(0, CPP, step=2)
        def _chunk(c):
            for b in range(2):
                cc = c + b
                pltpu.make_async_copy(hs.at[sidx.at[cc]], rows[b],
                                      sems[b]).wait()
                pltpu.sync_copy(rows[b], acc.at[didx.at[cc]], add=True)

                @pl.when(cc + 2 < CPP)
                def _refill():
                    pltpu.async_copy(hs.at[sidx.at[cc + 2]], rows[b], sems[b])

    plsc.subcore_barrier()
    pltpu.sync_copy(acc.at[pl.ds(sid * ROWS_PER_TILE, ROWS_PER_TILE)],
                    parts.at[cid, pl.ds(sid * ROWS_PER_TILE, ROWS_PER_TILE)])


def _t0_body(h_ref, o_ref):
    s = jnp.sum(h_ref[...], axis=0, keepdims=True)
    o_ref[...] = lax.rsqrt(s + 1.0)


_t0 = pl.pallas_call(
    _t0_body,
    grid=(1,),
    in_specs=[pl.BlockSpec((NTILES, NPAD), lambda i: (0, 0))],
    out_specs=pl.BlockSpec((1, NPAD), lambda i: (0, 0)),
    out_shape=jax.ShapeDtypeStruct((1, NPAD), jnp.float32),
)


def _t1_body(x_ref, w_ref, dv_ref, o_ref):
    o_ref[...] = jnp.dot(x_ref[...], w_ref[...],
                         preferred_element_type=jnp.float32) * dv_ref[...]


_t1 = pl.pallas_call(
    _t1_body,
    grid=(NBLK,),
    in_specs=[
        pl.BlockSpec((BLK, D), lambda i: (i, 0)),
        pl.BlockSpec((D, D), lambda i: (0, 0)),
        pl.BlockSpec((BLK, 1), lambda i: (i, 0)),
    ],
    out_specs=pl.BlockSpec((BLK, D), lambda i: (i, 0)),
    out_shape=jax.ShapeDtypeStruct((N, D), jnp.float32),
)


def _bn_relu(z, s1, s2, g, be):
    mean = s1 * (1.0 / N)
    var = s2 * (1.0 / N) - mean * mean
    rstd = lax.rsqrt(var + EPS_BN)
    return jnp.maximum((z - mean) * (rstd * g) + be, 0.0)


def _stats_phase(ph, i, p0_ref, p1_ref, dv_ref, zbuf, s1, s2):
    z = (p0_ref[0] + p1_ref[0]) * dv_ref[...]
    zbuf[pl.ds(i * BLK, BLK), :] = z
    c1 = jnp.sum(z, axis=0, keepdims=True)
    c2 = jnp.sum(z * z, axis=0, keepdims=True)

    @pl.when(i == 0)
    def _init():
        s1[...] = c1
        s2[...] = c2

    @pl.when(i > 0)
    def _acc():
        s1[...] += c1
        s2[...] += c2


def _t23_body(p0_ref, p1_ref, dv_ref, g_ref, be_ref, w_ref, o_ref,
              zbuf, s1, s2):
    ph = pl.program_id(0)
    i = pl.program_id(1)

    @pl.when(ph == 0)
    def _p0():
        _stats_phase(ph, i, p0_ref, p1_ref, dv_ref, zbuf, s1, s2)

    @pl.when(ph == 1)
    def _p1():
        h = _bn_relu(zbuf[pl.ds(i * BLK, BLK), :], s1[...], s2[...],
                     g_ref[...], be_ref[...])
        o_ref[...] = jnp.dot(h, w_ref[...],
                             preferred_element_type=jnp.float32) * dv_ref[...]


_t23 = pl.pallas_call(
    _t23_body,
    grid=(2, NBLK),
    in_specs=[
        pl.BlockSpec((1, BLK, D), lambda ph, i: (0, i * (1 - ph), 0)),
        pl.BlockSpec((1, BLK, D), lambda ph, i: (1, i * (1 - ph), 0)),
        pl.BlockSpec((BLK, 1), lambda ph, i: (i, 0)),
        pl.BlockSpec((1, D), lambda ph, i: (0, 0)),
        pl.BlockSpec((1, D), lambda ph, i: (0, 0)),
        pl.BlockSpec((D, D), lambda ph, i: (0, 0)),
    ],
    out_specs=pl.BlockSpec((BLK, D), lambda ph, i: (i * ph, 0)),
    out_shape=jax.ShapeDtypeStruct((N, D), jnp.float32),
    scratch_shapes=[
        pltpu.VMEM((N, D), jnp.float32),
        pltpu.VMEM((1, D), jnp.float32),
        pltpu.VMEM((1, D), jnp.float32),
    ],
)


def _t24_body(p0_ref, p1_ref, dv_ref, g_ref, be_ref, b_ref, wc_ref, bc_ref,
              feat_ref, pred_ref, zbuf, s1, s2, pool, cnt):
    ph = pl.program_id(0)
    i = pl.program_id(1)

    @pl.when(ph == 0)
    def _p0():
        _stats_phase(ph, i, p0_ref, p1_ref, dv_ref, zbuf, s1, s2)

    @pl.when(ph == 1)
    def _p1():
        h = _bn_relu(zbuf[pl.ds(i * BLK, BLK), :], s1[...], s2[...],
                     g_ref[...], be_ref[...])
        ids = lax.broadcasted_iota(jnp.int32, (BLK, G), 1)
        oh = (b_ref[...] == ids).astype(jnp.float32)
        ppart = lax.dot_general(oh, h, (((0,), (0,)), ((), ())),
                                preferred_element_type=jnp.float32)
        cpart = lax.dot_general(oh, jnp.ones((BLK, D), jnp.float32),
                                (((0,), (0,)), ((), ())),
                                preferred_element_type=jnp.float32)

        @pl.when(i == 0)
        def _init():
            pool[...] = ppart
            cnt[...] = cpart

        @pl.when(i > 0)
        def _acc():
            pool[...] += ppart
            cnt[...] += cpart

        @pl.when(i == NBLK - 1)
        def _fin():
            feat = pool[...] / jnp.maximum(cnt[...], 1.0)
            nrm = jnp.sqrt(jnp.sum(feat * feat, axis=1, keepdims=True))
            feat = feat / jnp.maximum(nrm, 1e-12)
            feat_ref[...] = feat
            pred_ref[...] = jnp.dot(
                feat, wc_ref[...],
                preferred_element_type=jnp.float32) + bc_ref[...]


_t24 = pl.pallas_call(
    _t24_body,
    grid=(2, NBLK),
    in_specs=[
        pl.BlockSpec((1, BLK, D), lambda ph, i: (0, i * (1 - ph), 0)),
        pl.BlockSpec((1, BLK, D), lambda ph, i: (1, i * (1 - ph), 0)),
        pl.BlockSpec((BLK, 1), lambda ph, i: (i, 0)),
        pl.BlockSpec((1, D), lambda ph, i: (0, 0)),
        pl.BlockSpec((1, D), lambda ph, i: (0, 0)),
        pl.BlockSpec((BLK, 1), lambda ph, i: (i, 0)),
        pl.BlockSpec((D, D), lambda ph, i: (0, 0)),
        pl.BlockSpec((1, D), lambda ph, i: (0, 0)),
    ],
    out_specs=[
        pl.BlockSpec((G, D), lambda ph, i: (0, 0)),
        pl.BlockSpec((G, D), lambda ph, i: (0, 0)),
    ],
    out_shape=[
        jax.ShapeDtypeStruct((G, D), jnp.float32),
        jax.ShapeDtypeStruct((G, D), jnp.float32),
    ],
    scratch_shapes=[
        pltpu.VMEM((N, D), jnp.float32),
        pltpu.VMEM((1, D), jnp.float32),
        pltpu.VMEM((1, D), jnp.float32),
        pltpu.VMEM((G, D), jnp.float32),
        pltpu.VMEM((G, D), jnp.float32),
    ],
)


def kernel(x, edge_index, batch, W1, b1, g1, be1, W2, b2, g2, be2,
           W3, b3, g3, be3, Wc, bc):
    f32 = jnp.float32
    pad = EPAD - E
    pad_i = jnp.arange(pad, dtype=jnp.int32)
    src1d = jnp.concatenate([edge_index[0], pad_i % N])
    dst1d = jnp.concatenate([edge_index[1], N + (pad_i % (NPAD - N))])
    src2d = src1d.reshape(EPAD // B, B)
    dst2d = dst1d.reshape(EPAD // B, B)
    zeros_np = jnp.zeros((NPAD, D), f32)

    hists = _sc_hist(dst1d)
    dinv_col = _t0(hists).reshape(NPAD, 1)

    g1r, be1r = g1.reshape(1, D), be1.reshape(1, D)
    g2r, be2r = g2.reshape(1, D), be2.reshape(1, D)
    g3r, be3r = g3.reshape(1, D), be3.reshape(1, D)
    batch_col = batch.reshape(N, 1)
    Wc_pad = jnp.zeros((D, D), f32).at[:, :C].set(Wc)
    bc_pad = jnp.zeros((1, D), f32).at[0, :C].set(bc)

    hs = _t1(x, W1, dinv_col)
    parts = _sc_scatter(src2d, dst2d, hs, zeros_np)
    hs = _t23(parts, parts, dinv_col, g1r, be1r, W2)
    parts = _sc_scatter(src2d, dst2d, hs, zeros_np)
    hs = _t23(parts, parts, dinv_col, g2r, be2r, W3)
    parts = _sc_scatter(src2d, dst2d, hs, zeros_np)
    feat, pred_pad = _t24(parts, parts, dinv_col, g3r, be3r, batch_col,
                          Wc_pad, bc_pad)
    return (pred_pad[:, :C], feat)
